# baseline (speedup 1.0000x reference)
import functools

import jax
import jax.numpy as jnp
from jax import lax
from jax.experimental import pallas as pl
from jax.experimental.pallas import tpu as pltpu
from jax.experimental.pallas import tpu_sc as plsc

_LANE = 128
_CAP = 1024
_CCAP = 64
_CGUARD = 48
_NCH = 31
_TCW = 8
_CW = _TCW * _LANE
_QSTR = 336


def _iota16():
    return lax.iota(jnp.int32, 16)


@functools.lru_cache(maxsize=None)
def _make_sc_stream_gather(num_rows, emb_dim, batch):
    mesh = plsc.VectorSubcoreMesh(core_axis_name="c", subcore_axis_name="s")
    nc, ns = mesh.num_cores, mesh.num_subcores
    n_tc = -(-num_rows // _LANE)
    dma_tc = _NCH * _TCW
    c0_total = ns * dma_tc
    out_rows = batch // 4

    @functools.partial(
        pl.kernel,
        mesh=mesh,
        out_type=jax.ShapeDtypeStruct((nc, out_rows, _LANE), jnp.float32),
        scratch_types=[
            pltpu.VMEM((batch,), jnp.int32),
            pltpu.VMEM((_CAP + 16,), jnp.int32),
            pltpu.VMEM((_CAP + 16,), jnp.int32),
            pltpu.VMEM((_CCAP + 16,), jnp.int32),
            pltpu.VMEM((_CCAP + 16,), jnp.int32),
            pltpu.VMEM((_CCAP,), jnp.int32),
            pltpu.VMEM((4 * _QSTR,), jnp.int32),
            pltpu.VMEM((4 * _QSTR,), jnp.int32),
            pltpu.VMEM((emb_dim, _CW), jnp.float32),
            pltpu.VMEM((emb_dim, _CW), jnp.float32),
            pltpu.VMEM((_CCAP, _LANE), jnp.float32),
            pltpu.VMEM_SHARED((out_rows, _LANE), jnp.float32),
            pltpu.SemaphoreType.DMA,
            pltpu.SemaphoreType.DMA,
        ],
        compiler_params=pltpu.CompilerParams(
            disable_bounds_checks=True, needs_layout_passes=False),
    )
    def gather_k(table_hbm, x_hbm, out_hbm, xv, rlist, ilist, clr, cli,
                 ridx, qlr, qli, buf0, buf1, rows, acc_sh, sem0, sem1):
        core = lax.axis_index("c")
        sub = lax.axis_index("s")
        rid = core * ns + sub
        k1 = rid - ns
        lo_tc = jnp.where(rid < ns, rid * dma_tc,
                          c0_total + 240 * k1 + jnp.minimum(k1, 5))
        w_tc = jnp.where(rid < ns, dma_tc,
                         jnp.where(k1 < 5, 241, 240))
        lo = lo_tc * _LANE
        hi = (lo_tc + w_tc) * _LANE
        dma_lo = jnp.minimum(lo_tc, n_tc - dma_tc) * _LANE

        zf16 = jnp.zeros((16,), jnp.float32)

        def zero_r(t, carry):
            rows[t >> 3, pl.ds((t & 7) * 16, 16)] = zf16
            return carry
        lax.fori_loop(0, _CCAP * 8, zero_r, 0)
        sh_rows = out_rows // ns
        for q in range(sh_rows // _CCAP):
            pltpu.sync_copy(
                rows, acc_sh.at[pl.ds(sub * sh_rows + q * _CCAP, _CCAP)])
        plsc.subcore_barrier()

        def prefetch(c, buf, sem):
            cc = jnp.minimum(c, _NCH - 1)
            return pltpu.async_copy(
                table_hbm.at[:, pl.ds(dma_lo + cc * _CW, _CW)], buf, sem)

        def buf_wait(buf, sem):
            pltpu.make_async_copy(
                table_hbm.at[:, pl.ds(0, _CW)], buf, sem).wait()

        prefetch(0, buf0, sem0)
        prefetch(1, buf1, sem1)

        pltpu.sync_copy(x_hbm, xv)

        def scan_body(i, cnt):
            r = xv[pl.ds(i * 16, 16)]
            cntv = jnp.full((16,), cnt, jnp.int32)
            m = (r >= lo) & (r < hi) & (cntv < _CAP - 16)
            ones = jnp.where(m, 1, 0)
            cs = jnp.cumsum(ones)
            dst = jnp.where(m, cntv + cs - 1, _CAP + _iota16())
            plsc.store_scatter(rlist, [dst], r)
            pos = _iota16() + i * 16
            plsc.store_scatter(ilist, [dst], pos)
            return cnt + cs[15]
        cnt = lax.fori_loop(0, batch // 16, scan_body, 0)
        nv1 = (cnt + 15) >> 4

        def quad_body(j, qc):
            slot = _iota16() + j * 16
            r = rlist[pl.ds(j * 16, 16)]
            p = ilist[pl.ds(j * 16, 16)]
            mb = slot < cnt
            qv = (r - dma_lo) >> 13
            out = []
            for q in range(4):
                cq = qc[q]
                cqv = jnp.full((16,), cq, jnp.int32)
                m = mb & (qv == q) & (cqv < _QSTR - 32)
                ones = jnp.where(m, 1, 0)
                cs = jnp.cumsum(ones)
                dst = jnp.where(m, q * _QSTR + cqv + cs - 1,
                                q * _QSTR + _QSTR - 16 + _iota16())
                plsc.store_scatter(qlr, [dst], r)
                plsc.store_scatter(qli, [dst], p)
                out.append(cq + cs[15])
            return tuple(out)
        qc = lax.fori_loop(0, nv1, quad_body, (0, 0, 0, 0))

        def process(c, buf):
            cwlo = dma_lo + c * _CW

            def zero_lists(j, carry2):
                z = jnp.zeros((16,), jnp.int32)
                cli[pl.ds(j * 16, 16)] = z
                clr[pl.ds(j * 16, 16)] = z
                ridx[pl.ds(j * 16, 16)] = z
                return carry2
            lax.fori_loop(0, _CCAP // 16, zero_lists, 0)

            q = c >> 3
            cq = jnp.where(q == 0, qc[0],
                           jnp.where(q == 1, qc[1],
                                     jnp.where(q == 2, qc[2], qc[3])))
            qbase = q * _QSTR

            def chunk_scan(j, ccnt):
                slot = _iota16() + j * 16
                r = qlr[pl.ds(qbase + j * 16, 16)]
                p = qli[pl.ds(qbase + j * 16, 16)]
                ccntv = jnp.full((16,), ccnt, jnp.int32)
                m = ((slot < cq) & (r >= cwlo) & (r < cwlo + _CW)
                     & (ccntv < _CGUARD))
                ones = jnp.where(m, 1, 0)
                cs = jnp.cumsum(ones)
                dst = jnp.where(m, ccntv + cs - 1, _CCAP + _iota16())
                plsc.store_scatter(clr, [dst], r)
                plsc.store_scatter(cli, [dst], p)
                return ccnt + cs[15]
            ccnt = lax.fori_loop(0, (cq + 15) >> 4, chunk_scan, 0)

            def extract(j, carry):
                slot = _iota16() + j * 16
                m = slot < ccnt
                r = clr[pl.ds(j * 16, 16)]
                p = cli[pl.ds(j * 16, 16)]
                ridx[pl.ds(j * 16, 16)] = jnp.where(m, p & 4095, 0)
                cloc = jnp.where(m, r - cwlo, 0)
                lane0 = (p >> 12) * emb_dim
                zf = jnp.zeros((16,), jnp.float32)
                for f in range(emb_dim):
                    fv = jnp.full((16,), f, jnp.int32)
                    vals = plsc.load_gather(buf, [fv, cloc])
                    plsc.store_scatter(rows, [slot, lane0 + f],
                                       jnp.where(m, vals, zf))
                return carry
            lax.fori_loop(0, (ccnt + 15) >> 4, extract, 0)

            pltpu.sync_copy(rows, acc_sh.at[ridx], add=True)

            def zero_used(j, carry2):
                slot = _iota16() + j * 16
                p = cli[pl.ds(j * 16, 16)]
                lane0 = (p >> 12) * emb_dim
                for f in range(emb_dim):
                    plsc.store_scatter(rows, [slot, lane0 + f], zf16)
                return carry2
            lax.fori_loop(0, (ccnt + 15) >> 4, zero_used, 0)

        def pair_body(t, carry):
            c0 = 2 * t
            c1 = 2 * t + 1
            buf_wait(buf0, sem0)
            process(c0, buf0)
            prefetch(c0 + 2, buf0, sem0)
            buf_wait(buf1, sem1)

            @pl.when(c1 < _NCH)
            def _():
                process(c1, buf1)

            prefetch(c1 + 2, buf1, sem1)
            return carry

        lax.fori_loop(0, (_NCH + 1) // 2, pair_body, 0)
        buf_wait(buf0, sem0)
        buf_wait(buf1, sem1)

        plsc.subcore_barrier()
        dst = out_hbm.at[core].at[pl.ds(sub * sh_rows, sh_rows)]
        pltpu.sync_copy(acc_sh.at[pl.ds(sub * sh_rows, sh_rows)], dst)

    return gather_k


def _mlp_body(e0_ref, e1_ref, w1_ref, b1_ref, w2_ref, b2_ref, wc_ref, bc_ref,
              feat_ref, logits_ref):
    k = pl.program_id(0) // 4
    e4 = e0_ref[...] + e1_ref[...]
    d = e4.shape[1] // 4
    e = jnp.where(
        k == 0, e4[:, 0:d],
        jnp.where(k == 1, e4[:, d:2 * d],
                  jnp.where(k == 2, e4[:, 2 * d:3 * d], e4[:, 3 * d:4 * d])))
    h = jnp.maximum(
        jnp.dot(e, w1_ref[...], preferred_element_type=jnp.float32)
        + b1_ref[...], 0.0)
    f = jnp.maximum(
        jnp.dot(h, w2_ref[...], preferred_element_type=jnp.float32)
        + b2_ref[...], 0.0)
    feat_ref[...] = f
    logits_ref[...] = (
        jnp.dot(f, wc_ref[...], preferred_element_type=jnp.float32)
        + bc_ref[...])


def _mlp_call(e0, e1, W1, b1, W2, b2, Wc_pad, bc_pad, bm, interpret=False):
    b = e0.shape[0] * 4
    emb_dim = W1.shape[0]
    hidden = W1.shape[1]
    out_dim = W2.shape[1]
    ncls = Wc_pad.shape[1]
    nb = b // bm
    return pl.pallas_call(
        _mlp_body,
        grid=(nb,),
        in_specs=[
            pl.BlockSpec((bm, _LANE), lambda i: (i % 4, 0)),
            pl.BlockSpec((bm, _LANE), lambda i: (i % 4, 0)),
            pl.BlockSpec((emb_dim, hidden), lambda i: (0, 0)),
            pl.BlockSpec((1, hidden), lambda i: (0, 0)),
            pl.BlockSpec((hidden, out_dim), lambda i: (0, 0)),
            pl.BlockSpec((1, out_dim), lambda i: (0, 0)),
            pl.BlockSpec((out_dim, ncls), lambda i: (0, 0)),
            pl.BlockSpec((1, ncls), lambda i: (0, 0)),
        ],
        out_specs=[
            pl.BlockSpec((bm, out_dim), lambda i: (i, 0)),
            pl.BlockSpec((bm, ncls), lambda i: (i, 0)),
        ],
        out_shape=[
            jax.ShapeDtypeStruct((b, out_dim), jnp.float32),
            jax.ShapeDtypeStruct((b, ncls), jnp.float32),
        ],
        interpret=interpret,
    )(e0, e1, W1, b1, W2, b2, Wc_pad, bc_pad)


def kernel(x, emb_table, W1, b1, W2, b2, Wc, bc):
    b = x.shape[0]
    num_rows, emb_dim = emb_table.shape
    gather_k = _make_sc_stream_gather(num_rows, emb_dim, b)
    e4 = gather_k(emb_table.T, x.astype(jnp.int32))

    ncls = Wc.shape[1]
    ncls_pad = 128
    Wc_pad = jnp.pad(Wc, ((0, 0), (0, ncls_pad - ncls)))
    bc_pad = jnp.pad(bc, (0, ncls_pad - ncls)).reshape(1, ncls_pad)
    feat, logits_pad = _mlp_call(
        e4[0], e4[1], W1, b1.reshape(1, -1), W2, b2.reshape(1, -1),
        Wc_pad, bc_pad, bm=1024)
    return logits_pad[:, :ncls], feat


# speedup vs baseline: 1.0009x; 1.0009x over previous
import functools

import jax
import jax.numpy as jnp
from jax import lax
from jax.experimental import pallas as pl
from jax.experimental.pallas import tpu as pltpu
from jax.experimental.pallas import tpu_sc as plsc

_LANE = 128
_CAP = 1024
_CCAP = 64
_CGUARD = 48
_NCH = 31
_TCW = 8
_CW = _TCW * _LANE
_QSTR = 336


def _iota16():
    return lax.iota(jnp.int32, 16)


@functools.lru_cache(maxsize=None)
def _make_sc_stream_gather(num_rows, emb_dim, batch):
    mesh = plsc.VectorSubcoreMesh(core_axis_name="c", subcore_axis_name="s")
    nc, ns = mesh.num_cores, mesh.num_subcores
    n_tc = -(-num_rows // _LANE)
    dma_tc = _NCH * _TCW
    c0_total = ns * dma_tc
    out_rows = batch // 4

    @functools.partial(
        pl.kernel,
        mesh=mesh,
        out_type=jax.ShapeDtypeStruct((nc, out_rows, _LANE), jnp.float32),
        scratch_types=[
            pltpu.VMEM((batch,), jnp.int32),
            pltpu.VMEM((_CAP + 16,), jnp.int32),
            pltpu.VMEM((_CAP + 16,), jnp.int32),
            pltpu.VMEM((_CCAP + 16,), jnp.int32),
            pltpu.VMEM((_CCAP + 16,), jnp.int32),
            pltpu.VMEM((_CCAP,), jnp.int32),
            pltpu.VMEM((4 * _QSTR,), jnp.int32),
            pltpu.VMEM((4 * _QSTR,), jnp.int32),
            pltpu.VMEM((emb_dim, _CW), jnp.float32),
            pltpu.VMEM((emb_dim, _CW), jnp.float32),
            pltpu.VMEM((_CCAP, _LANE), jnp.float32),
            pltpu.VMEM_SHARED((out_rows, _LANE), jnp.float32),
            pltpu.SemaphoreType.DMA,
            pltpu.SemaphoreType.DMA,
        ],
        compiler_params=pltpu.CompilerParams(
            disable_bounds_checks=True, needs_layout_passes=False),
    )
    def gather_k(table_hbm, x_hbm, out_hbm, xv, rlist, ilist, clr, cli,
                 ridx, qlr, qli, buf0, buf1, rows, acc_sh, sem0, sem1):
        core = lax.axis_index("c")
        sub = lax.axis_index("s")
        rid = core * ns + sub
        k1 = rid - ns
        lo_tc = jnp.where(rid < ns, rid * dma_tc,
                          c0_total + 240 * k1 + jnp.minimum(k1, 5))
        w_tc = jnp.where(rid < ns, dma_tc,
                         jnp.where(k1 < 5, 241, 240))
        lo = lo_tc * _LANE
        hi = (lo_tc + w_tc) * _LANE
        dma_lo = jnp.minimum(lo_tc, n_tc - dma_tc) * _LANE

        zf16 = jnp.zeros((16,), jnp.float32)

        def zero_r(t, carry):
            rows[t >> 3, pl.ds((t & 7) * 16, 16)] = zf16
            return carry
        lax.fori_loop(0, _CCAP * 8, zero_r, 0)
        sh_rows = out_rows // ns
        for q in range(sh_rows // _CCAP):
            pltpu.sync_copy(
                rows, acc_sh.at[pl.ds(sub * sh_rows + q * _CCAP, _CCAP)])
        plsc.subcore_barrier()

        def prefetch(c, buf, sem):
            cc = jnp.minimum(c, _NCH - 1)
            return pltpu.async_copy(
                table_hbm.at[:, pl.ds(dma_lo + cc * _CW, _CW)], buf, sem)

        def buf_wait(buf, sem):
            pltpu.make_async_copy(
                table_hbm.at[:, pl.ds(0, _CW)], buf, sem).wait()

        prefetch(0, buf0, sem0)
        prefetch(1, buf1, sem1)

        pltpu.sync_copy(x_hbm, xv)

        def scan_body(i, cnt):
            for u in range(2):
                r = xv[pl.ds(i * 32 + u * 16, 16)]
                cntv = jnp.full((16,), cnt, jnp.int32)
                m = (r >= lo) & (r < hi) & (cntv < _CAP - 16)
                ones = jnp.where(m, 1, 0)
                cs = jnp.cumsum(ones)
                dst = jnp.where(m, cntv + cs - 1, _CAP + _iota16())
                plsc.store_scatter(rlist, [dst], r)
                pos = _iota16() + i * 32 + u * 16
                plsc.store_scatter(ilist, [dst], pos)
                cnt = cnt + cs[15]
            return cnt
        cnt = lax.fori_loop(0, batch // 32, scan_body, 0)
        nv1 = (cnt + 15) >> 4

        def quad_body(j, qc):
            slot = _iota16() + j * 16
            r = rlist[pl.ds(j * 16, 16)]
            p = ilist[pl.ds(j * 16, 16)]
            mb = slot < cnt
            qv = (r - dma_lo) >> 13
            out = []
            for q in range(4):
                cq = qc[q]
                cqv = jnp.full((16,), cq, jnp.int32)
                m = mb & (qv == q) & (cqv < _QSTR - 32)
                ones = jnp.where(m, 1, 0)
                cs = jnp.cumsum(ones)
                dst = jnp.where(m, q * _QSTR + cqv + cs - 1,
                                q * _QSTR + _QSTR - 16 + _iota16())
                plsc.store_scatter(qlr, [dst], r)
                plsc.store_scatter(qli, [dst], p)
                out.append(cq + cs[15])
            return tuple(out)
        qc = lax.fori_loop(0, nv1, quad_body, (0, 0, 0, 0))

        def process(c, buf):
            cwlo = dma_lo + c * _CW

            def zero_lists(j, carry2):
                z = jnp.zeros((16,), jnp.int32)
                cli[pl.ds(j * 16, 16)] = z
                clr[pl.ds(j * 16, 16)] = z
                ridx[pl.ds(j * 16, 16)] = z
                return carry2
            lax.fori_loop(0, _CCAP // 16, zero_lists, 0)

            q = c >> 3
            cq = jnp.where(q == 0, qc[0],
                           jnp.where(q == 1, qc[1],
                                     jnp.where(q == 2, qc[2], qc[3])))
            qbase = q * _QSTR

            def chunk_scan(j, ccnt):
                slot = _iota16() + j * 16
                r = qlr[pl.ds(qbase + j * 16, 16)]
                p = qli[pl.ds(qbase + j * 16, 16)]
                ccntv = jnp.full((16,), ccnt, jnp.int32)
                m = ((slot < cq) & (r >= cwlo) & (r < cwlo + _CW)
                     & (ccntv < _CGUARD))
                ones = jnp.where(m, 1, 0)
                cs = jnp.cumsum(ones)
                dst = jnp.where(m, ccntv + cs - 1, _CCAP + _iota16())
                plsc.store_scatter(clr, [dst], r)
                plsc.store_scatter(cli, [dst], p)
                return ccnt + cs[15]
            ccnt = lax.fori_loop(0, (cq + 15) >> 4, chunk_scan, 0)

            def extract(j, carry):
                slot = _iota16() + j * 16
                m = slot < ccnt
                r = clr[pl.ds(j * 16, 16)]
                p = cli[pl.ds(j * 16, 16)]
                ridx[pl.ds(j * 16, 16)] = jnp.where(m, p & 4095, 0)
                cloc = jnp.where(m, r - cwlo, 0)
                lane0 = (p >> 12) * emb_dim
                zf = jnp.zeros((16,), jnp.float32)
                for f in range(emb_dim):
                    fv = jnp.full((16,), f, jnp.int32)
                    vals = plsc.load_gather(buf, [fv, cloc])
                    plsc.store_scatter(rows, [slot, lane0 + f],
                                       jnp.where(m, vals, zf))
                return carry
            lax.fori_loop(0, (ccnt + 15) >> 4, extract, 0)

            pltpu.sync_copy(rows, acc_sh.at[ridx], add=True)

            def zero_used(j, carry2):
                slot = _iota16() + j * 16
                p = cli[pl.ds(j * 16, 16)]
                lane0 = (p >> 12) * emb_dim
                for f in range(emb_dim):
                    plsc.store_scatter(rows, [slot, lane0 + f], zf16)
                return carry2
            lax.fori_loop(0, (ccnt + 15) >> 4, zero_used, 0)

        def pair_body(t, carry):
            c0 = 2 * t
            c1 = 2 * t + 1
            buf_wait(buf0, sem0)
            process(c0, buf0)
            prefetch(c0 + 2, buf0, sem0)
            buf_wait(buf1, sem1)

            @pl.when(c1 < _NCH)
            def _():
                process(c1, buf1)

            prefetch(c1 + 2, buf1, sem1)
            return carry

        lax.fori_loop(0, (_NCH + 1) // 2, pair_body, 0)
        buf_wait(buf0, sem0)
        buf_wait(buf1, sem1)

        plsc.subcore_barrier()
        dst = out_hbm.at[core].at[pl.ds(sub * sh_rows, sh_rows)]
        pltpu.sync_copy(acc_sh.at[pl.ds(sub * sh_rows, sh_rows)], dst)

    return gather_k


def _mlp_body(e0_ref, e1_ref, w1_ref, b1_ref, w2_ref, b2_ref, wc_ref, bc_ref,
              feat_ref, logits_ref):
    k = pl.program_id(0) // 4
    e4 = e0_ref[...] + e1_ref[...]
    d = e4.shape[1] // 4
    e = jnp.where(
        k == 0, e4[:, 0:d],
        jnp.where(k == 1, e4[:, d:2 * d],
                  jnp.where(k == 2, e4[:, 2 * d:3 * d], e4[:, 3 * d:4 * d])))
    h = jnp.maximum(
        jnp.dot(e, w1_ref[...], preferred_element_type=jnp.float32)
        + b1_ref[...], 0.0)
    f = jnp.maximum(
        jnp.dot(h, w2_ref[...], preferred_element_type=jnp.float32)
        + b2_ref[...], 0.0)
    feat_ref[...] = f
    logits_ref[...] = (
        jnp.dot(f, wc_ref[...], preferred_element_type=jnp.float32)
        + bc_ref[...])


def _mlp_call(e0, e1, W1, b1, W2, b2, Wc_pad, bc_pad, bm, interpret=False):
    b = e0.shape[0] * 4
    emb_dim = W1.shape[0]
    hidden = W1.shape[1]
    out_dim = W2.shape[1]
    ncls = Wc_pad.shape[1]
    nb = b // bm
    return pl.pallas_call(
        _mlp_body,
        grid=(nb,),
        in_specs=[
            pl.BlockSpec((bm, _LANE), lambda i: (i % 4, 0)),
            pl.BlockSpec((bm, _LANE), lambda i: (i % 4, 0)),
            pl.BlockSpec((emb_dim, hidden), lambda i: (0, 0)),
            pl.BlockSpec((1, hidden), lambda i: (0, 0)),
            pl.BlockSpec((hidden, out_dim), lambda i: (0, 0)),
            pl.BlockSpec((1, out_dim), lambda i: (0, 0)),
            pl.BlockSpec((out_dim, ncls), lambda i: (0, 0)),
            pl.BlockSpec((1, ncls), lambda i: (0, 0)),
        ],
        out_specs=[
            pl.BlockSpec((bm, out_dim), lambda i: (i, 0)),
            pl.BlockSpec((bm, ncls), lambda i: (i, 0)),
        ],
        out_shape=[
            jax.ShapeDtypeStruct((b, out_dim), jnp.float32),
            jax.ShapeDtypeStruct((b, ncls), jnp.float32),
        ],
        interpret=interpret,
    )(e0, e1, W1, b1, W2, b2, Wc_pad, bc_pad)


def kernel(x, emb_table, W1, b1, W2, b2, Wc, bc):
    b = x.shape[0]
    num_rows, emb_dim = emb_table.shape
    gather_k = _make_sc_stream_gather(num_rows, emb_dim, b)
    e4 = gather_k(emb_table.T, x.astype(jnp.int32))

    ncls = Wc.shape[1]
    ncls_pad = 128
    Wc_pad = jnp.pad(Wc, ((0, 0), (0, ncls_pad - ncls)))
    bc_pad = jnp.pad(bc, (0, ncls_pad - ncls)).reshape(1, ncls_pad)
    feat, logits_pad = _mlp_call(
        e4[0], e4[1], W1, b1.reshape(1, -1), W2, b2.reshape(1, -1),
        Wc_pad, bc_pad, bm=1024)
    return logits_pad[:, :ncls], feat


# speedup vs baseline: 1.0169x; 1.0160x over previous
import functools

import jax
import jax.numpy as jnp
from jax import lax
from jax.experimental import pallas as pl
from jax.experimental.pallas import tpu as pltpu
from jax.experimental.pallas import tpu_sc as plsc

_LANE = 128
_CAP = 1024
_CCAP = 64
_CGUARD = 48
_NCH = 31
_TCW = 8
_CW = _TCW * _LANE
_QSTR = 336


def _iota16():
    return lax.iota(jnp.int32, 16)


@functools.lru_cache(maxsize=None)
def _make_sc_stream_gather(num_rows, emb_dim, batch):
    mesh = plsc.VectorSubcoreMesh(core_axis_name="c", subcore_axis_name="s")
    nc, ns = mesh.num_cores, mesh.num_subcores
    n_tc = -(-num_rows // _LANE)
    dma_tc = _NCH * _TCW
    c0_total = ns * dma_tc
    out_rows = batch // 4

    @functools.partial(
        pl.kernel,
        mesh=mesh,
        out_type=jax.ShapeDtypeStruct((nc, out_rows, _LANE), jnp.float32),
        scratch_types=[
            pltpu.VMEM((batch,), jnp.int32),
            pltpu.VMEM((_CAP + 16,), jnp.int32),
            pltpu.VMEM((_CAP + 16,), jnp.int32),
            pltpu.VMEM((_CCAP + 16,), jnp.int32),
            pltpu.VMEM((_CCAP + 16,), jnp.int32),
            pltpu.VMEM((_CCAP,), jnp.int32),
            pltpu.VMEM((4 * _QSTR,), jnp.int32),
            pltpu.VMEM((4 * _QSTR,), jnp.int32),
            pltpu.VMEM((emb_dim, _CW), jnp.float32),
            pltpu.VMEM((emb_dim, _CW), jnp.float32),
            pltpu.VMEM((_CCAP, _LANE), jnp.float32),
            pltpu.VMEM_SHARED((out_rows, _LANE), jnp.float32),
            pltpu.SemaphoreType.DMA,
            pltpu.SemaphoreType.DMA,
        ],
        compiler_params=pltpu.CompilerParams(
            disable_bounds_checks=True, needs_layout_passes=False),
    )
    def gather_k(table_hbm, x_hbm, out_hbm, xv, rlist, ilist, clr, cli,
                 ridx, qlr, qli, buf0, buf1, rows, acc_sh, sem0, sem1):
        core = lax.axis_index("c")
        sub = lax.axis_index("s")
        rid = core * ns + sub
        k1 = rid - ns
        lo_tc = jnp.where(rid < ns, rid * dma_tc,
                          c0_total + 240 * k1 + jnp.minimum(k1, 5))
        w_tc = jnp.where(rid < ns, dma_tc,
                         jnp.where(k1 < 5, 241, 240))
        lo = lo_tc * _LANE
        hi = (lo_tc + w_tc) * _LANE
        dma_lo = jnp.minimum(lo_tc, n_tc - dma_tc) * _LANE

        zf16 = jnp.zeros((16,), jnp.float32)

        def zero_r(t, carry):
            rows[t >> 3, pl.ds((t & 7) * 16, 16)] = zf16
            return carry
        lax.fori_loop(0, _CCAP * 8, zero_r, 0)
        sh_rows = out_rows // ns
        for q in range(sh_rows // _CCAP):
            pltpu.sync_copy(
                rows, acc_sh.at[pl.ds(sub * sh_rows + q * _CCAP, _CCAP)])
        plsc.subcore_barrier()

        def prefetch(c, buf, sem):
            cc = jnp.minimum(c, _NCH - 1)
            return pltpu.async_copy(
                table_hbm.at[:, pl.ds(dma_lo + cc * _CW, _CW)], buf, sem)

        def buf_wait(buf, sem):
            pltpu.make_async_copy(
                table_hbm.at[:, pl.ds(0, _CW)], buf, sem).wait()

        prefetch(0, buf0, sem0)
        prefetch(1, buf1, sem1)

        pltpu.sync_copy(x_hbm, xv)

        def scan_body(i, cnt):
            for u in range(2):
                r = xv[pl.ds(i * 32 + u * 16, 16)]
                cntv = jnp.full((16,), cnt, jnp.int32)
                m = (r >= lo) & (r < hi) & (cntv < _CAP - 16)
                ones = jnp.where(m, 1, 0)
                cs = jnp.cumsum(ones)
                dst = jnp.where(m, cntv + cs - 1, _CAP + _iota16())
                plsc.store_scatter(rlist, [dst], r)
                pos = _iota16() + i * 32 + u * 16
                plsc.store_scatter(ilist, [dst], pos)
                cnt = cnt + cs[15]
            return cnt
        cnt = lax.fori_loop(0, batch // 32, scan_body, 0)
        nv1 = (cnt + 15) >> 4

        def quad_body(j, qc):
            slot = _iota16() + j * 16
            r = rlist[pl.ds(j * 16, 16)]
            p = ilist[pl.ds(j * 16, 16)]
            mb = slot < cnt
            qv = (r - dma_lo) >> 13
            out = []
            for q in range(4):
                cq = qc[q]
                cqv = jnp.full((16,), cq, jnp.int32)
                m = mb & (qv == q) & (cqv < _QSTR - 32)
                ones = jnp.where(m, 1, 0)
                cs = jnp.cumsum(ones)
                dst = jnp.where(m, q * _QSTR + cqv + cs - 1,
                                q * _QSTR + _QSTR - 16 + _iota16())
                plsc.store_scatter(qlr, [dst], r)
                plsc.store_scatter(qli, [dst], p)
                out.append(cq + cs[15])
            return tuple(out)
        qc = lax.fori_loop(0, nv1, quad_body, (0, 0, 0, 0))

        def process(c, buf):
            cwlo = dma_lo + c * _CW

            def zero_lists(j, carry2):
                z = jnp.zeros((16,), jnp.int32)
                cli[pl.ds(j * 16, 16)] = z
                clr[pl.ds(j * 16, 16)] = z
                ridx[pl.ds(j * 16, 16)] = z
                return carry2
            lax.fori_loop(0, _CCAP // 16, zero_lists, 0)

            q = c >> 3
            cq = jnp.where(q == 0, qc[0],
                           jnp.where(q == 1, qc[1],
                                     jnp.where(q == 2, qc[2], qc[3])))
            qbase = q * _QSTR

            def chunk_scan(j, ccnt):
                slot = _iota16() + j * 16
                r = qlr[pl.ds(qbase + j * 16, 16)]
                p = qli[pl.ds(qbase + j * 16, 16)]
                ccntv = jnp.full((16,), ccnt, jnp.int32)
                m = ((slot < cq) & (r >= cwlo) & (r < cwlo + _CW)
                     & (ccntv < _CGUARD))
                ones = jnp.where(m, 1, 0)
                cs = jnp.cumsum(ones)
                dst = jnp.where(m, ccntv + cs - 1, _CCAP + _iota16())
                plsc.store_scatter(clr, [dst], r)
                plsc.store_scatter(cli, [dst], p)
                return ccnt + cs[15]
            ccnt = lax.fori_loop(0, (cq + 15) >> 4, chunk_scan, 0)

            def extract(j, carry):
                slot = _iota16() + j * 16
                m = slot < ccnt
                r = clr[pl.ds(j * 16, 16)]
                p = cli[pl.ds(j * 16, 16)]
                ridx[pl.ds(j * 16, 16)] = jnp.where(m, p & 4095, 0)
                cloc = jnp.where(m, r - cwlo, 0)
                lane0 = (p >> 12) * emb_dim
                zf = jnp.zeros((16,), jnp.float32)
                for f in range(emb_dim):
                    fv = jnp.full((16,), f, jnp.int32)
                    vals = plsc.load_gather(buf, [fv, cloc])
                    plsc.store_scatter(rows, [slot, lane0 + f],
                                       jnp.where(m, vals, zf))
                return carry
            lax.fori_loop(0, (ccnt + 15) >> 4, extract, 0)

            pltpu.sync_copy(rows, acc_sh.at[ridx], add=True)

            def zero_used(j, carry2):
                slot = _iota16() + j * 16
                p = cli[pl.ds(j * 16, 16)]
                lane0 = (p >> 12) * emb_dim
                for f in range(emb_dim):
                    plsc.store_scatter(rows, [slot, lane0 + f], zf16)
                return carry2
            lax.fori_loop(0, (ccnt + 15) >> 4, zero_used, 0)

        def pair_body(t, carry):
            c0 = 2 * t
            c1 = 2 * t + 1
            buf_wait(buf0, sem0)
            process(c0, buf0)
            prefetch(c0 + 2, buf0, sem0)
            buf_wait(buf1, sem1)

            @pl.when(c1 < _NCH)
            def _():
                process(c1, buf1)

            prefetch(c1 + 2, buf1, sem1)
            return carry

        lax.fori_loop(0, (_NCH + 1) // 2, pair_body, 0)
        buf_wait(buf0, sem0)
        buf_wait(buf1, sem1)

        plsc.subcore_barrier()
        dst = out_hbm.at[core].at[pl.ds(sub * sh_rows, sh_rows)]
        pltpu.sync_copy(acc_sh.at[pl.ds(sub * sh_rows, sh_rows)], dst)

    return gather_k


def _mlp_body(e0_ref, e1_ref, s_ref, w1_ref, b1_ref, w2_ref, b2_ref,
              wc_ref, bc_ref, feat_ref, logits_ref):
    e4 = e0_ref[...] + e1_ref[...]
    e = jnp.dot(e4, s_ref[0], preferred_element_type=jnp.float32)
    h = jnp.maximum(
        jnp.dot(e, w1_ref[...], preferred_element_type=jnp.float32)
        + b1_ref[...], 0.0)
    f = jnp.maximum(
        jnp.dot(h, w2_ref[...], preferred_element_type=jnp.float32)
        + b2_ref[...], 0.0)
    feat_ref[...] = f
    logits_ref[...] = (
        jnp.dot(f, wc_ref[...], preferred_element_type=jnp.float32)
        + bc_ref[...])


def _mlp_call(e0, e1, W1, b1, W2, b2, Wc_pad, bc_pad, bm, interpret=False):
    b = e0.shape[0] * 4
    emb_dim = W1.shape[0]
    eye = jnp.eye(emb_dim, dtype=jnp.float32)
    S4 = jnp.stack([
        jnp.pad(eye, ((k * emb_dim, _LANE - (k + 1) * emb_dim), (0, 0)))
        for k in range(_LANE // emb_dim)])
    hidden = W1.shape[1]
    out_dim = W2.shape[1]
    ncls = Wc_pad.shape[1]
    nb = b // bm
    return pl.pallas_call(
        _mlp_body,
        grid=(nb,),
        in_specs=[
            pl.BlockSpec((bm, _LANE), lambda i: (i % 4, 0)),
            pl.BlockSpec((bm, _LANE), lambda i: (i % 4, 0)),
            pl.BlockSpec((1, _LANE, emb_dim), lambda i: (i // 4, 0, 0)),
            pl.BlockSpec((emb_dim, hidden), lambda i: (0, 0)),
            pl.BlockSpec((1, hidden), lambda i: (0, 0)),
            pl.BlockSpec((hidden, out_dim), lambda i: (0, 0)),
            pl.BlockSpec((1, out_dim), lambda i: (0, 0)),
            pl.BlockSpec((out_dim, ncls), lambda i: (0, 0)),
            pl.BlockSpec((1, ncls), lambda i: (0, 0)),
        ],
        out_specs=[
            pl.BlockSpec((bm, out_dim), lambda i: (i, 0)),
            pl.BlockSpec((bm, ncls), lambda i: (i, 0)),
        ],
        out_shape=[
            jax.ShapeDtypeStruct((b, out_dim), jnp.float32),
            jax.ShapeDtypeStruct((b, ncls), jnp.float32),
        ],
        interpret=interpret,
    )(e0, e1, S4, W1, b1, W2, b2, Wc_pad, bc_pad)


def kernel(x, emb_table, W1, b1, W2, b2, Wc, bc):
    b = x.shape[0]
    num_rows, emb_dim = emb_table.shape
    gather_k = _make_sc_stream_gather(num_rows, emb_dim, b)
    e4 = gather_k(emb_table.T, x.astype(jnp.int32))

    ncls = Wc.shape[1]
    ncls_pad = 128
    Wc_pad = jnp.pad(Wc, ((0, 0), (0, ncls_pad - ncls)))
    bc_pad = jnp.pad(bc, (0, ncls_pad - ncls)).reshape(1, ncls_pad)
    feat, logits_pad = _mlp_call(
        e4[0], e4[1], W1, b1.reshape(1, -1), W2, b2.reshape(1, -1),
        Wc_pad, bc_pad, bm=1024)
    return logits_pad[:, :ncls], feat


# speedup vs baseline: 1.0171x; 1.0002x over previous
import functools

import jax
import jax.numpy as jnp
from jax import lax
from jax.experimental import pallas as pl
from jax.experimental.pallas import tpu as pltpu
from jax.experimental.pallas import tpu_sc as plsc

_LANE = 128
_CAP = 1024
_CCAP = 64
_CGUARD = 48
_NCH = 31
_TCW = 8
_CW = _TCW * _LANE
_QSTR = 336


def _iota16():
    return lax.iota(jnp.int32, 16)


@functools.lru_cache(maxsize=None)
def _make_sc_stream_gather(num_rows, emb_dim, batch):
    mesh = plsc.VectorSubcoreMesh(core_axis_name="c", subcore_axis_name="s")
    nc, ns = mesh.num_cores, mesh.num_subcores
    n_tc = -(-num_rows // _LANE)
    dma_tc = _NCH * _TCW
    c0_total = ns * dma_tc
    out_rows = batch // 4

    @functools.partial(
        pl.kernel,
        mesh=mesh,
        out_type=jax.ShapeDtypeStruct((nc, out_rows, _LANE), jnp.float32),
        scratch_types=[
            pltpu.VMEM((batch,), jnp.int32),
            pltpu.VMEM((_CAP + 16,), jnp.int32),
            pltpu.VMEM((_CAP + 16,), jnp.int32),
            pltpu.VMEM((_CCAP + 16,), jnp.int32),
            pltpu.VMEM((_CCAP + 16,), jnp.int32),
            pltpu.VMEM((_CCAP,), jnp.int32),
            pltpu.VMEM((4 * _QSTR,), jnp.int32),
            pltpu.VMEM((4 * _QSTR,), jnp.int32),
            pltpu.VMEM((emb_dim, _CW), jnp.float32),
            pltpu.VMEM((emb_dim, _CW), jnp.float32),
            pltpu.VMEM((_CCAP, _LANE), jnp.float32),
            pltpu.VMEM_SHARED((out_rows, _LANE), jnp.float32),
            pltpu.SemaphoreType.DMA,
            pltpu.SemaphoreType.DMA,
        ],
        compiler_params=pltpu.CompilerParams(
            disable_bounds_checks=True, needs_layout_passes=False),
    )
    def gather_k(table_hbm, x_hbm, out_hbm, xv, rlist, ilist, clr, cli,
                 ridx, qlr, qli, buf0, buf1, rows, acc_sh, sem0, sem1):
        core = lax.axis_index("c")
        sub = lax.axis_index("s")
        rid = core * ns + sub
        k1 = rid - ns
        lo_tc = jnp.where(rid < ns, rid * dma_tc,
                          c0_total + 240 * k1 + jnp.minimum(k1, 5))
        w_tc = jnp.where(rid < ns, dma_tc,
                         jnp.where(k1 < 5, 241, 240))
        lo = lo_tc * _LANE
        hi = (lo_tc + w_tc) * _LANE
        dma_lo = jnp.minimum(lo_tc, n_tc - dma_tc) * _LANE

        zf16 = jnp.zeros((16,), jnp.float32)

        def zero_r(t, carry):
            rows[t >> 3, pl.ds((t & 7) * 16, 16)] = zf16
            return carry
        lax.fori_loop(0, _CCAP * 8, zero_r, 0)
        sh_rows = out_rows // ns
        for q in range(sh_rows // _CCAP):
            pltpu.sync_copy(
                rows, acc_sh.at[pl.ds(sub * sh_rows + q * _CCAP, _CCAP)])
        plsc.subcore_barrier()

        def prefetch(c, buf, sem):
            cc = jnp.minimum(c, _NCH - 1)
            return pltpu.async_copy(
                table_hbm.at[:, pl.ds(dma_lo + cc * _CW, _CW)], buf, sem)

        def buf_wait(buf, sem):
            pltpu.make_async_copy(
                table_hbm.at[:, pl.ds(0, _CW)], buf, sem).wait()

        prefetch(0, buf0, sem0)
        prefetch(1, buf1, sem1)

        pltpu.sync_copy(x_hbm, xv)

        def scan_body(i, cnt):
            for u in range(2):
                r = xv[pl.ds(i * 32 + u * 16, 16)]
                cntv = jnp.full((16,), cnt, jnp.int32)
                m = (r >= lo) & (r < hi) & (cntv < _CAP - 16)
                ones = jnp.where(m, 1, 0)
                cs = jnp.cumsum(ones)
                dst = jnp.where(m, cntv + cs - 1, _CAP + _iota16())
                plsc.store_scatter(rlist, [dst], r)
                pos = _iota16() + i * 32 + u * 16
                plsc.store_scatter(ilist, [dst], pos)
                cnt = cnt + cs[15]
            return cnt
        cnt = lax.fori_loop(0, batch // 32, scan_body, 0)
        nv1 = (cnt + 15) >> 4

        def quad_body(j, qc):
            slot = _iota16() + j * 16
            r = rlist[pl.ds(j * 16, 16)]
            p = ilist[pl.ds(j * 16, 16)]
            mb = slot < cnt
            qv = (r - dma_lo) >> 13
            out = []
            for q in range(4):
                cq = qc[q]
                cqv = jnp.full((16,), cq, jnp.int32)
                m = mb & (qv == q) & (cqv < _QSTR - 32)
                ones = jnp.where(m, 1, 0)
                cs = jnp.cumsum(ones)
                dst = jnp.where(m, q * _QSTR + cqv + cs - 1,
                                q * _QSTR + _QSTR - 16 + _iota16())
                plsc.store_scatter(qlr, [dst], r)
                plsc.store_scatter(qli, [dst], p)
                out.append(cq + cs[15])
            return tuple(out)
        qc = lax.fori_loop(0, nv1, quad_body, (0, 0, 0, 0))

        def process(c, buf):
            cwlo = dma_lo + c * _CW

            def zero_lists(j, carry2):
                z = jnp.zeros((16,), jnp.int32)
                cli[pl.ds(j * 16, 16)] = z
                clr[pl.ds(j * 16, 16)] = z
                ridx[pl.ds(j * 16, 16)] = z
                return carry2
            lax.fori_loop(0, _CCAP // 16, zero_lists, 0)

            q = c >> 3
            cq = jnp.where(q == 0, qc[0],
                           jnp.where(q == 1, qc[1],
                                     jnp.where(q == 2, qc[2], qc[3])))
            qbase = q * _QSTR

            def chunk_scan(j, ccnt):
                slot = _iota16() + j * 16
                r = qlr[pl.ds(qbase + j * 16, 16)]
                p = qli[pl.ds(qbase + j * 16, 16)]
                ccntv = jnp.full((16,), ccnt, jnp.int32)
                m = ((slot < cq) & (r >= cwlo) & (r < cwlo + _CW)
                     & (ccntv < _CGUARD))
                ones = jnp.where(m, 1, 0)
                cs = jnp.cumsum(ones)
                dst = jnp.where(m, ccntv + cs - 1, _CCAP + _iota16())
                plsc.store_scatter(clr, [dst], r)
                plsc.store_scatter(cli, [dst], p)
                return ccnt + cs[15]
            ccnt = lax.fori_loop(0, (cq + 15) >> 4, chunk_scan, 0)

            def extract(j, carry):
                slot = _iota16() + j * 16
                m = slot < ccnt
                r = clr[pl.ds(j * 16, 16)]
                p = cli[pl.ds(j * 16, 16)]
                ridx[pl.ds(j * 16, 16)] = jnp.where(m, p & 4095, 0)
                cloc = jnp.where(m, r - cwlo, 0)
                lane0 = (p >> 12) * emb_dim
                zf = jnp.zeros((16,), jnp.float32)
                for f in range(emb_dim):
                    fv = jnp.full((16,), f, jnp.int32)
                    vals = plsc.load_gather(buf, [fv, cloc])
                    plsc.store_scatter(rows, [slot, lane0 + f],
                                       jnp.where(m, vals, zf))
                return carry
            lax.fori_loop(0, (ccnt + 15) >> 4, extract, 0)

            pltpu.sync_copy(rows, acc_sh.at[ridx], add=True)

            def zero_used(j, carry2):
                slot = _iota16() + j * 16
                p = cli[pl.ds(j * 16, 16)]
                lane0 = (p >> 12) * emb_dim
                for f in range(emb_dim):
                    plsc.store_scatter(rows, [slot, lane0 + f], zf16)
                return carry2
            lax.fori_loop(0, (ccnt + 15) >> 4, zero_used, 0)

        def pair_body(t, carry):
            c0 = 2 * t
            c1 = 2 * t + 1
            buf_wait(buf0, sem0)
            process(c0, buf0)
            prefetch(c0 + 2, buf0, sem0)
            buf_wait(buf1, sem1)

            @pl.when(c1 < _NCH)
            def _():
                process(c1, buf1)

            prefetch(c1 + 2, buf1, sem1)
            return carry

        lax.fori_loop(0, (_NCH + 1) // 2, pair_body, 0)
        buf_wait(buf0, sem0)
        buf_wait(buf1, sem1)

        plsc.subcore_barrier()
        dst = out_hbm.at[core].at[pl.ds(sub * sh_rows, sh_rows)]
        pltpu.sync_copy(acc_sh.at[pl.ds(sub * sh_rows, sh_rows)], dst)

    return gather_k


def _mlp_body(e0_ref, e1_ref, s_ref, w1_ref, b1_ref, w2_ref, b2_ref,
              wc_ref, bc_ref, feat_ref, logits_ref):
    e4 = e0_ref[...] + e1_ref[...]
    e = jnp.dot(e4, s_ref[0], preferred_element_type=jnp.float32)
    h = jnp.maximum(
        jnp.dot(e.astype(jnp.bfloat16), w1_ref[...].astype(jnp.bfloat16),
                preferred_element_type=jnp.float32)
        + b1_ref[...], 0.0)
    f = jnp.maximum(
        jnp.dot(h.astype(jnp.bfloat16), w2_ref[...].astype(jnp.bfloat16),
                preferred_element_type=jnp.float32)
        + b2_ref[...], 0.0)
    feat_ref[...] = f
    logits_ref[...] = (
        jnp.dot(f.astype(jnp.bfloat16), wc_ref[...].astype(jnp.bfloat16),
                preferred_element_type=jnp.float32)
        + bc_ref[...])


def _mlp_call(e0, e1, W1, b1, W2, b2, Wc_pad, bc_pad, bm, interpret=False):
    b = e0.shape[0] * 4
    emb_dim = W1.shape[0]
    eye = jnp.eye(emb_dim, dtype=jnp.float32)
    S4 = jnp.stack([
        jnp.pad(eye, ((k * emb_dim, _LANE - (k + 1) * emb_dim), (0, 0)))
        for k in range(_LANE // emb_dim)])
    hidden = W1.shape[1]
    out_dim = W2.shape[1]
    ncls = Wc_pad.shape[1]
    nb = b // bm
    return pl.pallas_call(
        _mlp_body,
        grid=(nb,),
        in_specs=[
            pl.BlockSpec((bm, _LANE), lambda i: (i % 4, 0)),
            pl.BlockSpec((bm, _LANE), lambda i: (i % 4, 0)),
            pl.BlockSpec((1, _LANE, emb_dim), lambda i: (i // 4, 0, 0)),
            pl.BlockSpec((emb_dim, hidden), lambda i: (0, 0)),
            pl.BlockSpec((1, hidden), lambda i: (0, 0)),
            pl.BlockSpec((hidden, out_dim), lambda i: (0, 0)),
            pl.BlockSpec((1, out_dim), lambda i: (0, 0)),
            pl.BlockSpec((out_dim, ncls), lambda i: (0, 0)),
            pl.BlockSpec((1, ncls), lambda i: (0, 0)),
        ],
        out_specs=[
            pl.BlockSpec((bm, out_dim), lambda i: (i, 0)),
            pl.BlockSpec((bm, ncls), lambda i: (i, 0)),
        ],
        out_shape=[
            jax.ShapeDtypeStruct((b, out_dim), jnp.float32),
            jax.ShapeDtypeStruct((b, ncls), jnp.float32),
        ],
        interpret=interpret,
    )(e0, e1, S4, W1, b1, W2, b2, Wc_pad, bc_pad)


def kernel(x, emb_table, W1, b1, W2, b2, Wc, bc):
    b = x.shape[0]
    num_rows, emb_dim = emb_table.shape
    gather_k = _make_sc_stream_gather(num_rows, emb_dim, b)
    e4 = gather_k(emb_table.T, x.astype(jnp.int32))

    ncls = Wc.shape[1]
    ncls_pad = 128
    Wc_pad = jnp.pad(Wc, ((0, 0), (0, ncls_pad - ncls)))
    bc_pad = jnp.pad(bc, (0, ncls_pad - ncls)).reshape(1, ncls_pad)
    feat, logits_pad = _mlp_call(
        e4[0], e4[1], W1, b1.reshape(1, -1), W2, b2.reshape(1, -1),
        Wc_pad, bc_pad, bm=1024)
    return logits_pad[:, :ncls], feat


# speedup vs baseline: 1.0353x; 1.0179x over previous
import functools

import jax
import jax.numpy as jnp
from jax import lax
from jax.experimental import pallas as pl
from jax.experimental.pallas import tpu as pltpu
from jax.experimental.pallas import tpu_sc as plsc

_LANE = 128
_CAP = 1024
_CCAP = 64
_CGUARD = 48
_NCH = 31
_TCW = 8
_CW = _TCW * _LANE
_QSTR = 336


def _iota16():
    return lax.iota(jnp.int32, 16)


@functools.lru_cache(maxsize=None)
def _make_sc_stream_gather(num_rows, emb_dim, batch):
    mesh = plsc.VectorSubcoreMesh(core_axis_name="c", subcore_axis_name="s")
    nc, ns = mesh.num_cores, mesh.num_subcores
    n_tc = -(-num_rows // _LANE)
    dma_tc = _NCH * _TCW
    c0_total = ns * dma_tc
    out_rows = batch // 4

    @functools.partial(
        pl.kernel,
        mesh=mesh,
        out_type=jax.ShapeDtypeStruct((nc, out_rows, _LANE), jnp.float32),
        scratch_types=[
            pltpu.VMEM((batch,), jnp.int32),
            pltpu.VMEM((_CAP + 16,), jnp.int32),
            pltpu.VMEM((_CAP + 16,), jnp.int32),
            pltpu.VMEM((_CCAP + 16,), jnp.int32),
            pltpu.VMEM((_CCAP + 16,), jnp.int32),
            pltpu.VMEM((_CCAP,), jnp.int32),
            pltpu.VMEM((4 * _QSTR,), jnp.int32),
            pltpu.VMEM((4 * _QSTR,), jnp.int32),
            pltpu.VMEM((emb_dim, _CW), jnp.float32),
            pltpu.VMEM((emb_dim, _CW), jnp.float32),
            pltpu.VMEM((_CCAP, _LANE), jnp.float32),
            pltpu.VMEM_SHARED((out_rows, _LANE), jnp.float32),
            pltpu.SemaphoreType.DMA,
            pltpu.SemaphoreType.DMA,
        ],
        compiler_params=pltpu.CompilerParams(
            disable_bounds_checks=True, needs_layout_passes=False),
    )
    def gather_k(table_hbm, x_hbm, out_hbm, xv, rlist, ilist, clr, cli,
                 ridx, qlr, qli, buf0, buf1, rows, acc_sh, sem0, sem1):
        core = lax.axis_index("c")
        sub = lax.axis_index("s")
        rid = core * ns + sub
        k1 = rid - ns
        lo_tc = jnp.where(rid < ns, rid * dma_tc,
                          c0_total + 240 * k1 + jnp.minimum(k1, 5))
        w_tc = jnp.where(rid < ns, dma_tc,
                         jnp.where(k1 < 5, 241, 240))
        lo = lo_tc * _LANE
        hi = (lo_tc + w_tc) * _LANE
        dma_lo = jnp.minimum(lo_tc, n_tc - dma_tc) * _LANE

        zf16 = jnp.zeros((16,), jnp.float32)

        def zero_r(t, carry):
            rows[t >> 3, pl.ds((t & 7) * 16, 16)] = zf16
            return carry
        lax.fori_loop(0, _CCAP * 8, zero_r, 0)
        sh_rows = out_rows // ns
        for q in range(sh_rows // _CCAP):
            pltpu.sync_copy(
                rows, acc_sh.at[pl.ds(sub * sh_rows + q * _CCAP, _CCAP)])
        plsc.subcore_barrier()

        def prefetch(c, buf, sem):
            cc = jnp.minimum(c, _NCH - 1)
            return pltpu.async_copy(
                table_hbm.at[:, pl.ds(dma_lo + cc * _CW, _CW)], buf, sem)

        def buf_wait(buf, sem):
            pltpu.make_async_copy(
                table_hbm.at[:, pl.ds(0, _CW)], buf, sem).wait()

        prefetch(0, buf0, sem0)
        prefetch(1, buf1, sem1)

        pltpu.sync_copy(x_hbm, xv)

        def scan_body(i, cnt):
            for u in range(2):
                r = xv[pl.ds(i * 32 + u * 16, 16)]
                cntv = jnp.full((16,), cnt, jnp.int32)
                m = (r >= lo) & (r < hi) & (cntv < _CAP - 16)
                ones = jnp.where(m, 1, 0)
                cs = jnp.cumsum(ones)
                dst = jnp.where(m, cntv + cs - 1, _CAP + _iota16())
                plsc.store_scatter(rlist, [dst], r)
                pos = _iota16() + i * 32 + u * 16
                plsc.store_scatter(ilist, [dst], pos)
                cnt = cnt + cs[15]
            return cnt
        cnt = lax.fori_loop(0, batch // 32, scan_body, 0)
        nv1 = (cnt + 15) >> 4

        def quad_body(j, qc):
            slot = _iota16() + j * 16
            r = rlist[pl.ds(j * 16, 16)]
            p = ilist[pl.ds(j * 16, 16)]
            mb = slot < cnt
            qv = (r - dma_lo) >> 13
            out = []
            for q in range(4):
                cq = qc[q]
                cqv = jnp.full((16,), cq, jnp.int32)
                m = mb & (qv == q) & (cqv < _QSTR - 32)
                ones = jnp.where(m, 1, 0)
                cs = jnp.cumsum(ones)
                dst = jnp.where(m, q * _QSTR + cqv + cs - 1,
                                q * _QSTR + _QSTR - 16 + _iota16())
                plsc.store_scatter(qlr, [dst], r)
                plsc.store_scatter(qli, [dst], p)
                out.append(cq + cs[15])
            return tuple(out)
        qc = lax.fori_loop(0, nv1, quad_body, (0, 0, 0, 0))

        def process(c, buf):
            cwlo = dma_lo + c * _CW

            def zero_lists(j, carry2):
                z = jnp.zeros((16,), jnp.int32)
                cli[pl.ds(j * 16, 16)] = z
                clr[pl.ds(j * 16, 16)] = z
                ridx[pl.ds(j * 16, 16)] = z
                return carry2
            lax.fori_loop(0, _CCAP // 16, zero_lists, 0)

            q = c >> 3
            cq = jnp.where(q == 0, qc[0],
                           jnp.where(q == 1, qc[1],
                                     jnp.where(q == 2, qc[2], qc[3])))
            qbase = q * _QSTR

            def chunk_scan(j, ccnt):
                slot = _iota16() + j * 16
                r = qlr[pl.ds(qbase + j * 16, 16)]
                p = qli[pl.ds(qbase + j * 16, 16)]
                ccntv = jnp.full((16,), ccnt, jnp.int32)
                m = ((slot < cq) & (r >= cwlo) & (r < cwlo + _CW)
                     & (ccntv < _CGUARD))
                ones = jnp.where(m, 1, 0)
                cs = jnp.cumsum(ones)
                dst = jnp.where(m, ccntv + cs - 1, _CCAP + _iota16())
                plsc.store_scatter(clr, [dst], r)
                plsc.store_scatter(cli, [dst], p)
                return ccnt + cs[15]
            ccnt = lax.fori_loop(0, (cq + 15) >> 4, chunk_scan, 0)

            def extract(j, carry):
                slot = _iota16() + j * 16
                m = slot < ccnt
                r = clr[pl.ds(j * 16, 16)]
                p = cli[pl.ds(j * 16, 16)]
                ridx[pl.ds(j * 16, 16)] = jnp.where(m, p & 4095, 0)
                cloc = jnp.where(m, r - cwlo, 0)
                lane0 = (p >> 12) * emb_dim
                zf = jnp.zeros((16,), jnp.float32)
                for f in range(emb_dim):
                    fv = jnp.full((16,), f, jnp.int32)
                    vals = plsc.load_gather(buf, [fv, cloc])
                    plsc.store_scatter(rows, [slot, lane0 + f],
                                       jnp.where(m, vals, zf))
                return carry
            lax.fori_loop(0, (ccnt + 15) >> 4, extract, 0)

            pltpu.sync_copy(rows, acc_sh.at[ridx], add=True)

            def zero_used(j, carry2):
                slot = _iota16() + j * 16
                p = cli[pl.ds(j * 16, 16)]
                lane0 = (p >> 12) * emb_dim
                for f in range(emb_dim):
                    plsc.store_scatter(rows, [slot, lane0 + f], zf16)
                return carry2
            lax.fori_loop(0, (ccnt + 15) >> 4, zero_used, 0)

        def pair_body(t, carry):
            c0 = 2 * t
            c1 = 2 * t + 1
            buf_wait(buf0, sem0)
            process(c0, buf0)
            prefetch(c0 + 2, buf0, sem0)
            buf_wait(buf1, sem1)

            @pl.when(c1 < _NCH)
            def _():
                process(c1, buf1)

            prefetch(c1 + 2, buf1, sem1)
            return carry

        lax.fori_loop(0, (_NCH + 1) // 2, pair_body, 0)
        buf_wait(buf0, sem0)
        buf_wait(buf1, sem1)

        plsc.subcore_barrier()
        dst = out_hbm.at[core].at[pl.ds(sub * sh_rows, sh_rows)]
        pltpu.sync_copy(acc_sh.at[pl.ds(sub * sh_rows, sh_rows)], dst)

    return gather_k


def _mlp_body(e0_ref, e1_ref, s_ref, w1_ref, b1_ref, w2_ref, b2_ref,
              wc_ref, bc_ref, feat_ref, logits_ref):
    e4 = e0_ref[...] + e1_ref[...]
    e = jnp.dot(e4, s_ref[0], preferred_element_type=jnp.float32)
    h = jnp.maximum(
        jnp.dot(e, w1_ref[...], preferred_element_type=jnp.float32)
        + b1_ref[...], 0.0)
    f = jnp.maximum(
        jnp.dot(h, w2_ref[...], preferred_element_type=jnp.float32)
        + b2_ref[...], 0.0)
    feat_ref[...] = f
    logits_ref[...] = (
        jnp.dot(f, wc_ref[...], preferred_element_type=jnp.float32)
        + bc_ref[...])


def _mlp_call(e0, e1, W1, b1, W2, b2, Wc_pad, bc_pad, bm, interpret=False):
    b = e0.shape[0] * 4
    emb_dim = W1.shape[0]
    eye = jnp.eye(emb_dim, dtype=jnp.float32)
    S4 = jnp.stack([
        jnp.pad(eye, ((k * emb_dim, _LANE - (k + 1) * emb_dim), (0, 0)))
        for k in range(_LANE // emb_dim)])
    hidden = W1.shape[1]
    out_dim = W2.shape[1]
    ncls = Wc_pad.shape[1]
    nb = b // bm
    return pl.pallas_call(
        _mlp_body,
        grid=(nb,),
        in_specs=[
            pl.BlockSpec((bm, _LANE), lambda i: (i % 2, 0)),
            pl.BlockSpec((bm, _LANE), lambda i: (i % 2, 0)),
            pl.BlockSpec((1, _LANE, emb_dim), lambda i: (i // 2, 0, 0)),
            pl.BlockSpec((emb_dim, hidden), lambda i: (0, 0)),
            pl.BlockSpec((1, hidden), lambda i: (0, 0)),
            pl.BlockSpec((hidden, out_dim), lambda i: (0, 0)),
            pl.BlockSpec((1, out_dim), lambda i: (0, 0)),
            pl.BlockSpec((out_dim, ncls), lambda i: (0, 0)),
            pl.BlockSpec((1, ncls), lambda i: (0, 0)),
        ],
        out_specs=[
            pl.BlockSpec((bm, out_dim), lambda i: (i, 0)),
            pl.BlockSpec((bm, ncls), lambda i: (i, 0)),
        ],
        out_shape=[
            jax.ShapeDtypeStruct((b, out_dim), jnp.float32),
            jax.ShapeDtypeStruct((b, ncls), jnp.float32),
        ],
        interpret=interpret,
    )(e0, e1, S4, W1, b1, W2, b2, Wc_pad, bc_pad)


def kernel(x, emb_table, W1, b1, W2, b2, Wc, bc):
    b = x.shape[0]
    num_rows, emb_dim = emb_table.shape
    gather_k = _make_sc_stream_gather(num_rows, emb_dim, b)
    e4 = gather_k(emb_table.T, x.astype(jnp.int32))

    ncls = Wc.shape[1]
    ncls_pad = 128
    Wc_pad = jnp.pad(Wc, ((0, 0), (0, ncls_pad - ncls)))
    bc_pad = jnp.pad(bc, (0, ncls_pad - ncls)).reshape(1, ncls_pad)
    feat, logits_pad = _mlp_call(
        e4[0], e4[1], W1, b1.reshape(1, -1), W2, b2.reshape(1, -1),
        Wc_pad, bc_pad, bm=2048)
    return logits_pad[:, :ncls], feat


# speedup vs baseline: 1.0420x; 1.0065x over previous
import functools

import jax
import jax.numpy as jnp
from jax import lax
from jax.experimental import pallas as pl
from jax.experimental.pallas import tpu as pltpu
from jax.experimental.pallas import tpu_sc as plsc

_LANE = 128
_CAP = 1024
_CCAP = 64
_CGUARD = 48
_NCH = 31
_TCW = 8
_CW = _TCW * _LANE
_QSTR = 336


def _iota16():
    return lax.iota(jnp.int32, 16)


@functools.lru_cache(maxsize=None)
def _make_sc_stream_gather(num_rows, emb_dim, batch):
    mesh = plsc.VectorSubcoreMesh(core_axis_name="c", subcore_axis_name="s")
    nc, ns = mesh.num_cores, mesh.num_subcores
    n_tc = -(-num_rows // _LANE)
    dma_tc = _NCH * _TCW
    c0_total = ns * dma_tc
    out_rows = batch // 4

    @functools.partial(
        pl.kernel,
        mesh=mesh,
        out_type=jax.ShapeDtypeStruct((nc, out_rows, _LANE), jnp.float32),
        scratch_types=[
            pltpu.VMEM((batch,), jnp.int32),
            pltpu.VMEM((_CAP + 16,), jnp.int32),
            pltpu.VMEM((_CAP + 16,), jnp.int32),
            pltpu.VMEM((_CCAP + 16,), jnp.int32),
            pltpu.VMEM((_CCAP + 16,), jnp.int32),
            pltpu.VMEM((_CCAP,), jnp.int32),
            pltpu.VMEM((4 * _QSTR,), jnp.int32),
            pltpu.VMEM((4 * _QSTR,), jnp.int32),
            pltpu.VMEM((emb_dim, _CW), jnp.float32),
            pltpu.VMEM((emb_dim, _CW), jnp.float32),
            pltpu.VMEM((_CCAP, _LANE), jnp.float32),
            pltpu.VMEM_SHARED((out_rows, _LANE), jnp.float32),
            pltpu.SemaphoreType.DMA,
            pltpu.SemaphoreType.DMA,
        ],
        compiler_params=pltpu.CompilerParams(
            disable_bounds_checks=True, needs_layout_passes=False),
    )
    def gather_k(table_hbm, x_hbm, out_hbm, xv, rlist, ilist, clr, cli,
                 ridx, qlr, qli, buf0, buf1, rows, acc_sh, sem0, sem1):
        core = lax.axis_index("c")
        sub = lax.axis_index("s")
        rid = core * ns + sub
        k1 = rid - ns
        lo_tc = jnp.where(rid < ns, rid * dma_tc,
                          c0_total + 240 * k1 + jnp.minimum(k1, 5))
        w_tc = jnp.where(rid < ns, dma_tc,
                         jnp.where(k1 < 5, 241, 240))
        lo = lo_tc * _LANE
        hi = (lo_tc + w_tc) * _LANE
        dma_lo = jnp.minimum(lo_tc, n_tc - dma_tc) * _LANE

        zf16 = jnp.zeros((16,), jnp.float32)

        def zero_r(t, carry):
            rows[t >> 3, pl.ds((t & 7) * 16, 16)] = zf16
            return carry
        lax.fori_loop(0, _CCAP * 8, zero_r, 0)
        sh_rows = out_rows // ns
        for q in range(sh_rows // _CCAP):
            pltpu.sync_copy(
                rows, acc_sh.at[pl.ds(sub * sh_rows + q * _CCAP, _CCAP)])
        plsc.subcore_barrier()

        def prefetch(c, buf, sem):
            cc = jnp.minimum(c, _NCH - 1)
            return pltpu.async_copy(
                table_hbm.at[:, pl.ds(dma_lo + cc * _CW, _CW)], buf, sem)

        def buf_wait(buf, sem):
            pltpu.make_async_copy(
                table_hbm.at[:, pl.ds(0, _CW)], buf, sem).wait()

        prefetch(0, buf0, sem0)
        prefetch(1, buf1, sem1)

        pltpu.sync_copy(x_hbm, xv)

        def scan_body(i, cnt):
            for u in range(2):
                r = xv[pl.ds(i * 32 + u * 16, 16)]
                cntv = jnp.full((16,), cnt, jnp.int32)
                m = (r >= lo) & (r < hi) & (cntv < _CAP - 16)
                ones = jnp.where(m, 1, 0)
                cs = jnp.cumsum(ones)
                dst = jnp.where(m, cntv + cs - 1, _CAP + _iota16())
                plsc.store_scatter(rlist, [dst], r)
                pos = _iota16() + i * 32 + u * 16
                plsc.store_scatter(ilist, [dst], pos)
                cnt = cnt + cs[15]
            return cnt
        cnt = lax.fori_loop(0, batch // 32, scan_body, 0)
        nv1 = (cnt + 15) >> 4

        def quad_body(j, qc):
            slot = _iota16() + j * 16
            r = rlist[pl.ds(j * 16, 16)]
            p = ilist[pl.ds(j * 16, 16)]
            mb = slot < cnt
            qv = (r - dma_lo) >> 13
            out = []
            for q in range(4):
                cq = qc[q]
                cqv = jnp.full((16,), cq, jnp.int32)
                m = mb & (qv == q) & (cqv < _QSTR - 32)
                ones = jnp.where(m, 1, 0)
                cs = jnp.cumsum(ones)
                dst = jnp.where(m, q * _QSTR + cqv + cs - 1,
                                q * _QSTR + _QSTR - 16 + _iota16())
                plsc.store_scatter(qlr, [dst], r)
                plsc.store_scatter(qli, [dst], p)
                out.append(cq + cs[15])
            return tuple(out)
        qc = lax.fori_loop(0, nv1, quad_body, (0, 0, 0, 0))

        def process(c, buf):
            cwlo = dma_lo + c * _CW

            def zero_lists(j, carry2):
                z = jnp.zeros((16,), jnp.int32)
                cli[pl.ds(j * 16, 16)] = z
                clr[pl.ds(j * 16, 16)] = z
                ridx[pl.ds(j * 16, 16)] = z
                return carry2
            lax.fori_loop(0, _CCAP // 16, zero_lists, 0)

            q = c >> 3
            cq = jnp.where(q == 0, qc[0],
                           jnp.where(q == 1, qc[1],
                                     jnp.where(q == 2, qc[2], qc[3])))
            qbase = q * _QSTR

            def chunk_scan(j, ccnt):
                slot = _iota16() + j * 16
                r = qlr[pl.ds(qbase + j * 16, 16)]
                p = qli[pl.ds(qbase + j * 16, 16)]
                ccntv = jnp.full((16,), ccnt, jnp.int32)
                m = ((slot < cq) & (r >= cwlo) & (r < cwlo + _CW)
                     & (ccntv < _CGUARD))
                ones = jnp.where(m, 1, 0)
                cs = jnp.cumsum(ones)
                dst = jnp.where(m, ccntv + cs - 1, _CCAP + _iota16())
                plsc.store_scatter(clr, [dst], r)
                plsc.store_scatter(cli, [dst], p)
                return ccnt + cs[15]
            ccnt = lax.fori_loop(0, (cq + 15) >> 4, chunk_scan, 0)

            def extract(j, carry):
                slot = _iota16() + j * 16
                m = slot < ccnt
                r = clr[pl.ds(j * 16, 16)]
                p = cli[pl.ds(j * 16, 16)]
                ridx[pl.ds(j * 16, 16)] = jnp.where(m, p & 4095, 0)
                cloc = jnp.where(m, r - cwlo, 0)
                lane0 = (p >> 12) * emb_dim
                zf = jnp.zeros((16,), jnp.float32)
                for f in range(emb_dim):
                    fv = jnp.full((16,), f, jnp.int32)
                    vals = plsc.load_gather(buf, [fv, cloc])
                    plsc.store_scatter(rows, [slot, lane0 + f],
                                       jnp.where(m, vals, zf))
                return carry
            lax.fori_loop(0, (ccnt + 15) >> 4, extract, 0)

            pltpu.sync_copy(rows, acc_sh.at[ridx], add=True)

            def zero_used(j, carry2):
                slot = _iota16() + j * 16
                p = cli[pl.ds(j * 16, 16)]
                lane0 = (p >> 12) * emb_dim
                for f in range(emb_dim):
                    plsc.store_scatter(rows, [slot, lane0 + f], zf16)
                return carry2
            lax.fori_loop(0, (ccnt + 15) >> 4, zero_used, 0)

        def pair_body(t, carry):
            c0 = 2 * t
            c1 = 2 * t + 1
            buf_wait(buf0, sem0)
            process(c0, buf0)
            prefetch(c0 + 2, buf0, sem0)
            buf_wait(buf1, sem1)

            @pl.when(c1 < _NCH)
            def _():
                process(c1, buf1)

            prefetch(c1 + 2, buf1, sem1)
            return carry

        lax.fori_loop(0, (_NCH + 1) // 2, pair_body, 0)
        buf_wait(buf0, sem0)
        buf_wait(buf1, sem1)

        plsc.subcore_barrier()
        dst = out_hbm.at[core].at[pl.ds(sub * sh_rows, sh_rows)]
        pltpu.sync_copy(acc_sh.at[pl.ds(sub * sh_rows, sh_rows)], dst)

    return gather_k


def _mlp_body(e0_ref, e1_ref, s_ref, w1_ref, b1_ref, w2_ref, b2_ref,
              wc_ref, bc_ref, feat_ref, logits_ref):
    e4 = e0_ref[...] + e1_ref[...]
    e = jnp.dot(e4, s_ref[0], preferred_element_type=jnp.float32)
    h = jnp.maximum(
        jnp.dot(e, w1_ref[...], preferred_element_type=jnp.float32)
        + b1_ref[...], 0.0)
    f = jnp.maximum(
        jnp.dot(h, w2_ref[...], preferred_element_type=jnp.float32)
        + b2_ref[...], 0.0)
    feat_ref[...] = f
    logits_ref[...] = (
        jnp.dot(f, wc_ref[...], preferred_element_type=jnp.float32)
        + bc_ref[...])


def _mlp_call(e0, e1, W1, b1, W2, b2, Wc_pad, bc_pad, bm, interpret=False):
    b = e0.shape[0] * 4
    emb_dim = W1.shape[0]
    eye = jnp.eye(emb_dim, dtype=jnp.float32)
    S4 = jnp.stack([
        jnp.pad(eye, ((k * emb_dim, _LANE - (k + 1) * emb_dim), (0, 0)))
        for k in range(_LANE // emb_dim)])
    hidden = W1.shape[1]
    out_dim = W2.shape[1]
    ncls = Wc_pad.shape[1]
    nb = b // bm
    return pl.pallas_call(
        _mlp_body,
        grid=(nb,),
        in_specs=[
            pl.BlockSpec((bm, _LANE), lambda i: (0, 0)),
            pl.BlockSpec((bm, _LANE), lambda i: (0, 0)),
            pl.BlockSpec((1, _LANE, emb_dim), lambda i: (i, 0, 0)),
            pl.BlockSpec((emb_dim, hidden), lambda i: (0, 0)),
            pl.BlockSpec((1, hidden), lambda i: (0, 0)),
            pl.BlockSpec((hidden, out_dim), lambda i: (0, 0)),
            pl.BlockSpec((1, out_dim), lambda i: (0, 0)),
            pl.BlockSpec((out_dim, ncls), lambda i: (0, 0)),
            pl.BlockSpec((1, ncls), lambda i: (0, 0)),
        ],
        out_specs=[
            pl.BlockSpec((bm, out_dim), lambda i: (i, 0)),
            pl.BlockSpec((bm, ncls), lambda i: (i, 0)),
        ],
        out_shape=[
            jax.ShapeDtypeStruct((b, out_dim), jnp.float32),
            jax.ShapeDtypeStruct((b, ncls), jnp.float32),
        ],
        interpret=interpret,
    )(e0, e1, S4, W1, b1, W2, b2, Wc_pad, bc_pad)


def kernel(x, emb_table, W1, b1, W2, b2, Wc, bc):
    b = x.shape[0]
    num_rows, emb_dim = emb_table.shape
    gather_k = _make_sc_stream_gather(num_rows, emb_dim, b)
    e4 = gather_k(emb_table.T, x.astype(jnp.int32))

    ncls = Wc.shape[1]
    ncls_pad = 128
    Wc_pad = jnp.pad(Wc, ((0, 0), (0, ncls_pad - ncls)))
    bc_pad = jnp.pad(bc, (0, ncls_pad - ncls)).reshape(1, ncls_pad)
    feat, logits_pad = _mlp_call(
        e4[0], e4[1], W1, b1.reshape(1, -1), W2, b2.reshape(1, -1),
        Wc_pad, bc_pad, bm=4096)
    return logits_pad[:, :ncls], feat
